# baseline (speedup 1.0000x reference)
import jax
import jax.numpy as jnp
from jax.experimental import pallas as pl
from jax.experimental.pallas import tpu as pltpu

_N_Y = 8000
_N_X = 2000
_N = _N_Y + _N_X
_NFEAT = 128
_NHID_LOW = 256
_NHID_HIGH = 64
_NCLASS = 40

_BM = 200
_P = _N // _BM


def _fused_body(
    xe_ref, y_ref, fc1w_ref, fc1b_ref, gc1w_ref, gc1b_ref, gc2w_ref, gc2b_ref,
    f_ref, c_ref,
    out_ref, yemb_ref,
    s1t_scr, s2_scr,
):
    i = pl.program_id(0)

    @pl.when(i == 0)
    def _prep():
        gc1w = gc1w_ref[...]
        s1t_scr[:, :_N_Y] = jax.lax.dot_general(
            gc1w, y_ref[...],
            dimension_numbers=(((0,), (1,)), ((), ())),
            preferred_element_type=jnp.float32,
        )
        xnt = (
            jax.lax.dot_general(
                fc1w_ref[...], xe_ref[...],
                dimension_numbers=(((1,), (1,)), ((), ())),
                preferred_element_type=jnp.float32,
            )
            + fc1b_ref[...]
        )
        s1t_scr[:, _N_Y:] = jax.lax.dot_general(
            gc1w, xnt,
            dimension_numbers=(((0,), (0,)), ((), ())),
            preferred_element_type=jnp.float32,
        )

    @pl.when(i < _P)
    def _phase1():
        yemb = jnp.maximum(
            jax.lax.dot_general(
                f_ref[...], s1t_scr[...],
                dimension_numbers=(((1,), (1,)), ((), ())),
                preferred_element_type=jnp.float32,
            )
            + gc1b_ref[...],
            0.0,
        )
        yemb_ref[...] = yemb
        s2_scr[pl.ds(i * _BM, _BM), :] = jnp.dot(
            yemb, gc2w_ref[...], preferred_element_type=jnp.float32
        )

    @pl.when(i >= _P)
    def _phase2():
        logits = (
            jnp.dot(c_ref[...], s2_scr[...], preferred_element_type=jnp.float32)
            + gc2b_ref[...]
        )
        m = jnp.max(logits, axis=1, keepdims=True)
        lse = jnp.log(jnp.sum(jnp.exp(logits - m), axis=1, keepdims=True)) + m
        out_ref[...] = logits - lse


def kernel(X_embedding, Y, F_tilde, C_tilde, fc1_W, fc1_b, gc1_W, gc1_b, gc2_W, gc2_b):
    fc1_bt = fc1_b.reshape(_NFEAT, 1)
    gc1_b2 = gc1_b.reshape(1, _NHID_HIGH)
    gc2_b2 = gc2_b.reshape(1, _NCLASS)

    const = lambda i: (0, 0)
    f_idx = lambda i: (jnp.minimum(i, _P - 1), 0)
    c_idx = lambda i: (jnp.maximum(i - _P, 0), 0)

    out, yemb = pl.pallas_call(
        _fused_body,
        grid=(2 * _P,),
        in_specs=[
            pl.BlockSpec((_N_X, _NHID_LOW), const),
            pl.BlockSpec((_N_Y, _NFEAT), const),
            pl.BlockSpec((_NFEAT, _NHID_LOW), const),
            pl.BlockSpec((_NFEAT, 1), const),
            pl.BlockSpec((_NFEAT, _NHID_HIGH), const),
            pl.BlockSpec((1, _NHID_HIGH), const),
            pl.BlockSpec((_NHID_HIGH, _NCLASS), const),
            pl.BlockSpec((1, _NCLASS), const),
            pl.BlockSpec((_BM, _N), f_idx),
            pl.BlockSpec((_BM, _N), c_idx),
        ],
        out_specs=[
            pl.BlockSpec((_BM, _NCLASS), c_idx),
            pl.BlockSpec((_BM, _NHID_HIGH), f_idx),
        ],
        out_shape=[
            jax.ShapeDtypeStruct((_N, _NCLASS), jnp.float32),
            jax.ShapeDtypeStruct((_N, _NHID_HIGH), jnp.float32),
        ],
        scratch_shapes=[
            pltpu.VMEM((_NHID_HIGH, _N), jnp.float32),
            pltpu.VMEM((_N, _NCLASS), jnp.float32),
        ],
    )(
        X_embedding, Y, fc1_W, fc1_bt, gc1_W, gc1_b2, gc2_W, gc2_b2,
        F_tilde, C_tilde,
    )

    return (out, yemb)
